# baseline (speedup 1.0000x reference)
import functools

import jax
import jax.numpy as jnp
from jax import lax
from jax.experimental import pallas as pl
from jax.experimental.pallas import tpu as pltpu
from jax.experimental.pallas import tpu_sc as plsc

N = 50000
E = 800000
DIM = 64
N_TYPES = 100
N_GRAPHS = 2000
CUTOFF = 5.0
N_CENTERS = 5
N_CONV = 3

CH_N = 2000
CHE = 4000
IDX_W = 100
IDX_H = 4
K_EDGE = IDX_W * IDX_H
E4 = E // 4
G_PAD = 2048
NT = 16
EPT = E // NT
CZ = 2000
LN2 = 0.6931471805599453

_INTERPRET = False


def _sp05(x):
    a = 0.5 * x
    return 2.0 * (jnp.maximum(a, 0.0) + jnp.log(1.0 + jnp.exp(-jnp.abs(a))))


def _split3(a):
    a1 = a.astype(jnp.bfloat16).astype(jnp.float32)
    r = a - a1
    a2 = r.astype(jnp.bfloat16).astype(jnp.float32)
    return a1, a2, r - a2


def _fdot(a, b):
    a1, a2, a3 = _split3(a)
    b1, b2, b3 = _split3(b)
    d = lambda x, y: jnp.dot(x, y, preferred_element_type=jnp.float32)
    return (d(a1, b1) + (d(a1, b2) + d(a2, b1))
            + (d(a1, b3) + d(a2, b2) + d(a3, b1)))


def _pickdot(oh, b):
    b1, b2, b3 = _split3(b)
    d = lambda x, y: jnp.dot(x, y, preferred_element_type=jnp.float32)
    return d(oh, b1) + d(oh, b2) + d(oh, b3)


def _embed_block(nt_ref, emb_ref, w1_ref, node_ref, nna_ref, nnb_ref):
    nt = nt_ref[0, 0, :]
    oh = (nt[:, None] == lax.broadcasted_iota(jnp.int32, (CH_N, 128), 1))
    node = _pickdot(oh.astype(jnp.float32), emb_ref[...])
    node_ref[...] = node
    nn = _fdot(node, w1_ref[...])
    nna_ref[...] = nn[:, :32]
    nnb_ref[...] = nn[:, 32:]


def _embed(node_type, emb_pad, w1):
    nt3 = node_type.reshape(N // CH_N, 1, CH_N)
    return pl.pallas_call(
        _embed_block,
        grid=(N // CH_N,),
        in_specs=[pl.BlockSpec((1, 1, CH_N), lambda i: (i, 0, 0)),
                  pl.BlockSpec((128, DIM), lambda i: (0, 0)),
                  pl.BlockSpec((DIM, DIM), lambda i: (0, 0))],
        out_specs=[pl.BlockSpec((CH_N, DIM), lambda i: (i, 0)),
                   pl.BlockSpec((CH_N, 32), lambda i: (i, 0)),
                   pl.BlockSpec((CH_N, 32), lambda i: (i, 0))],
        out_shape=[jax.ShapeDtypeStruct((N, DIM), jnp.float32),
                   jax.ShapeDtypeStruct((N, 32), jnp.float32),
                   jax.ShapeDtypeStruct((N, 32), jnp.float32)],
        interpret=_INTERPRET,
    )(nt3, emb_pad, w1)


def _filter_block(d_ref, w1_ref, b1_ref, w2_ref, b2_ref, ha_ref, hb_ref):
    d = d_ref[0, 0, :][:, None]
    gap = CUTOFF / (N_CENTERS - 1)
    c8 = lax.broadcasted_iota(jnp.int32, (CHE, 8), 1).astype(jnp.float32) * gap
    rbf = jnp.exp(-(d - c8) ** 2 * (1.0 / gap))
    t = _fdot(rbf, w1_ref[...]) \
        + b1_ref[...]
    h1 = _sp05(t)
    h = _fdot(h1, w2_ref[...]) \
        + b2_ref[...]
    q = CHE // 4
    ha_ref[...] = jnp.concatenate(
        [h[j * q:(j + 1) * q, :32] for j in range(4)], axis=1)
    hb_ref[...] = jnp.concatenate(
        [h[j * q:(j + 1) * q, 32:] for j in range(4)], axis=1)


def _filter_layer(d3, w1p, b1, w2, b2):
    return pl.pallas_call(
        _filter_block,
        grid=(E // CHE,),
        in_specs=[pl.BlockSpec((1, 1, CHE), lambda i: (i, 0, 0)),
                  pl.BlockSpec((8, DIM), lambda i: (0, 0)),
                  pl.BlockSpec((1, DIM), lambda i: (0, 0)),
                  pl.BlockSpec((DIM, DIM), lambda i: (0, 0)),
                  pl.BlockSpec((1, DIM), lambda i: (0, 0))],
        out_specs=[pl.BlockSpec((CHE // 4, 128), lambda i: (i, 0))] * 2,
        out_shape=[jax.ShapeDtypeStruct((E4, 128), jnp.float32)] * 2,
        interpret=_INTERPRET,
    )(d3, w1p, b1, w2, b2)


def _update_block(cfa_ref, cfb_ref, node_ref, w2_ref, b2_ref, w3_ref, b3_ref,
                  w1n_ref, nodeo_ref, nna_ref, nnb_ref):
    cf = jnp.concatenate([cfa_ref[...], cfb_ref[...]], axis=1)
    cf1 = _fdot(cf, w2_ref[...]) \
        + b2_ref[...]
    s = _sp05(cf1)
    nd = node_ref[...] + _fdot(s, w3_ref[...]) + b3_ref[...]
    nodeo_ref[...] = nd
    nn = _fdot(nd, w1n_ref[...])
    nna_ref[...] = nn[:, :32]
    nnb_ref[...] = nn[:, 32:]


def _update(cf_a, cf_b, node, w2, b2, w3, b3, w1n):
    return pl.pallas_call(
        _update_block,
        grid=(N // CH_N,),
        in_specs=[pl.BlockSpec((CH_N, 32), lambda i: (i, 0)),
                  pl.BlockSpec((CH_N, 32), lambda i: (i, 0)),
                  pl.BlockSpec((CH_N, DIM), lambda i: (i, 0)),
                  pl.BlockSpec((DIM, DIM), lambda i: (0, 0)),
                  pl.BlockSpec((1, DIM), lambda i: (0, 0)),
                  pl.BlockSpec((DIM, DIM), lambda i: (0, 0)),
                  pl.BlockSpec((1, DIM), lambda i: (0, 0)),
                  pl.BlockSpec((DIM, DIM), lambda i: (0, 0))],
        out_specs=[pl.BlockSpec((CH_N, DIM), lambda i: (i, 0)),
                   pl.BlockSpec((CH_N, 32), lambda i: (i, 0)),
                   pl.BlockSpec((CH_N, 32), lambda i: (i, 0))],
        out_shape=[jax.ShapeDtypeStruct((N, DIM), jnp.float32),
                   jax.ShapeDtypeStruct((N, 32), jnp.float32),
                   jax.ShapeDtypeStruct((N, 32), jnp.float32)],
        interpret=_INTERPRET,
    )(cf_a, cf_b, node, w2, b2, w3, b3, w1n)


def _head_block(cfa_ref, cfb_ref, node_ref, w2_ref, b2_ref, w3_ref, b3_ref,
                d1w_ref, d1b_ref, d2w_ref, d2b_ref, res_ref):
    cf = jnp.concatenate([cfa_ref[...], cfb_ref[...]], axis=1)
    cf1 = _fdot(cf, w2_ref[...]) \
        + b2_ref[...]
    s = _sp05(cf1)
    nd = node_ref[...] + _fdot(s, w3_ref[...]) + b3_ref[...]
    atom = _fdot(nd, d1w_ref[...]) \
        + d1b_ref[...]
    a2 = jnp.maximum(atom, 0.0) + jnp.log(1.0 + jnp.exp(-jnp.abs(atom))) - LN2
    r = jnp.sum(a2 * d2w_ref[...], axis=1)[:, None] + d2b_ref[0, 0]
    lane = lax.broadcasted_iota(jnp.int32, (CH_N, 16), 1)
    res_ref[...] = jnp.where(lane == 0, r,
                             jnp.where(lane == 1, 1.0, 0.0))


def _head(cf_a, cf_b, node, w2, b2, w3, b3, d1w, d1b, d2w, d2b):
    return pl.pallas_call(
        _head_block,
        grid=(N // CH_N,),
        in_specs=[pl.BlockSpec((CH_N, 32), lambda i: (i, 0)),
                  pl.BlockSpec((CH_N, 32), lambda i: (i, 0)),
                  pl.BlockSpec((CH_N, DIM), lambda i: (i, 0)),
                  pl.BlockSpec((DIM, DIM), lambda i: (0, 0)),
                  pl.BlockSpec((1, DIM), lambda i: (0, 0)),
                  pl.BlockSpec((DIM, DIM), lambda i: (0, 0)),
                  pl.BlockSpec((1, DIM), lambda i: (0, 0)),
                  pl.BlockSpec((DIM, DIM), lambda i: (0, 0)),
                  pl.BlockSpec((1, DIM), lambda i: (0, 0)),
                  pl.BlockSpec((1, DIM), lambda i: (0, 0)),
                  pl.BlockSpec((1, 1), lambda i: (0, 0))],
        out_specs=pl.BlockSpec((CH_N, 16), lambda i: (i, 0)),
        out_shape=jax.ShapeDtypeStruct((N, 16), jnp.float32),
        interpret=_INTERPRET,
    )(cf_a, cf_b, node, w2, b2, w3, b3, d1w, d1b, d2w, d2b)


def _conv_sc(nn_a, nn_b, h_a, h_b, src2, dst2, zeros_nc):
    mesh = plsc.VectorSubcoreMesh(core_axis_name="c", subcore_axis_name="s")

    @functools.partial(
        pl.kernel,
        out_type=[jax.ShapeDtypeStruct((N, 32), jnp.float32),
                  jax.ShapeDtypeStruct((N, 32), jnp.float32)],
        mesh=mesh,
        scratch_types=[
            pltpu.VMEM((IDX_H, IDX_W), jnp.int32),
            pltpu.VMEM((IDX_H, IDX_W), jnp.int32),
            pltpu.VMEM((IDX_H, IDX_W, 32), jnp.float32),
            pltpu.VMEM((IDX_W, 128), jnp.float32),
            pltpu.VMEM_SHARED((N, 32), jnp.float32),
            pltpu.SemaphoreType.DMA,
        ],
        compiler_params=pltpu.CompilerParams(use_tc_tiling_on_sc=False),
    )
    def k(nna_hbm, nnb_hbm, ha_hbm, hb_hbm, src_hbm, dst_hbm, z_hbm,
          outa_hbm, outb_hbm, sidx, didx, gv, hv, acc, sem):
        c = lax.axis_index("c")
        s = lax.axis_index("s")

        def run(nn_ref, h_ref, out_ref):
            def zchunk(j, _):
                ci = s + j * NT

                @pl.when(ci < N // CZ)
                def _():
                    pltpu.sync_copy(z_hbm.at[pl.ds(ci * CZ, CZ)],
                                    acc.at[pl.ds(ci * CZ, CZ)])

                return 0

            lax.fori_loop(0, (N // CZ + NT - 1) // NT, zchunk, 0)
            plsc.subcore_barrier()

            def chunk(i, _):
                g = s + i * NT
                pltpu.sync_copy(src_hbm.at[pl.ds(IDX_H * g, IDX_H)], sidx)
                pltpu.sync_copy(dst_hbm.at[pl.ds(IDX_H * g, IDX_H)], didx)
                pltpu.sync_copy(h_ref.at[pl.ds(IDX_W * g, IDX_W)], hv)
                for a in range(IDX_H):
                    pltpu.sync_copy(nn_ref.at[sidx.at[a]], gv.at[a])

                def mulrow(r, _):
                    for a in range(IDX_H):
                        for t in range(2):
                            gv[a, r, pl.ds(t * 16, 16)] = (
                                gv[a, r, pl.ds(t * 16, 16)]
                                * hv[r, pl.ds(a * 32 + t * 16, 16)])
                    return 0

                lax.fori_loop(0, IDX_W, mulrow, 0)
                for a in range(IDX_H):
                    pltpu.sync_copy(gv.at[a], acc.at[didx.at[a]], add=True)
                return 0

            lax.fori_loop(0, E // (K_EDGE * NT), chunk, 0)
            plsc.subcore_barrier()

            def ochunk(j, _):
                ci = s + j * NT

                @pl.when(ci < N // CZ)
                def _():
                    pltpu.sync_copy(acc.at[pl.ds(ci * CZ, CZ)],
                                    out_ref.at[pl.ds(ci * CZ, CZ)])

                return 0

            lax.fori_loop(0, (N // CZ + NT - 1) // NT, ochunk, 0)

        @pl.when(c == 0)
        def _():
            run(nna_hbm, ha_hbm, outa_hbm)

        @pl.when(c == 1)
        def _():
            run(nnb_hbm, hb_hbm, outb_hbm)

    return k(nn_a, nn_b, h_a, h_b, src2, dst2, zeros_nc)


def _pool_sc(res3, gid2, zeros_pool):
    mesh = plsc.VectorSubcoreMesh(core_axis_name="c", subcore_axis_name="s")
    nchunk = N // K_EDGE
    iters = (nchunk + NT - 1) // NT
    gpt = G_PAD // NT

    @functools.partial(
        pl.kernel,
        out_type=jax.ShapeDtypeStruct((G_PAD, 16), jnp.float32),
        mesh=mesh,
        scratch_types=[
            pltpu.VMEM((IDX_H, IDX_W), jnp.int32),
            pltpu.VMEM((IDX_H, IDX_W, 16), jnp.float32),
            pltpu.VMEM_SHARED((G_PAD, 16), jnp.float32),
        ],
        compiler_params=pltpu.CompilerParams(use_tc_tiling_on_sc=False),
    )
    def k(res_hbm, gid_hbm, z_hbm, out_hbm, gidx, vv, acc):
        c = lax.axis_index("c")
        s = lax.axis_index("s")

        @pl.when(c == 0)
        def _():
            pltpu.sync_copy(z_hbm.at[pl.ds(s * gpt, gpt)],
                            acc.at[pl.ds(s * gpt, gpt)])
            plsc.subcore_barrier()

            def it(j, _):
                ci = s + j * NT

                @pl.when(ci < nchunk)
                def _():
                    rb = ci * IDX_H
                    pltpu.sync_copy(gid_hbm.at[pl.ds(rb, IDX_H)], gidx)
                    pltpu.sync_copy(res_hbm.at[pl.ds(rb, IDX_H)], vv)
                    for a in range(IDX_H):
                        pltpu.sync_copy(vv.at[a], acc.at[gidx.at[a]],
                                        add=True)

                return 0

            lax.fori_loop(0, iters, it, 0)
            plsc.subcore_barrier()
            pltpu.sync_copy(acc.at[pl.ds(s * gpt, gpt)],
                            out_hbm.at[pl.ds(s * gpt, gpt)])

    return k(res3, gid2, zeros_pool)


def kernel(node_type, edge_index, distance, graph_ids, params):
    p = params
    seg = CHE // 4 // IDX_W
    def _perm(v):
        return v.reshape(E // CHE, 4, seg, IDX_W).transpose(0, 2, 1, 3) \
                .reshape(E // IDX_W, IDX_W)
    src2 = _perm(edge_index[0])
    dst2 = _perm(edge_index[1])
    emb_pad = jnp.zeros((128, DIM), jnp.float32).at[:N_TYPES].set(p["emb"])
    d3 = distance.reshape(E // CHE, 1, CHE)
    hs = []
    for i in range(N_CONV):
        ci = p["conv%d" % i]
        w1p = jnp.zeros((8, DIM), jnp.float32).at[:N_CENTERS].set(
            ci["cf_W1"])
        hs.extend(_filter_layer(d3, w1p, ci["cf_b1"].reshape(1, DIM),
                                ci["cf_W2"], ci["cf_b2"].reshape(1, DIM)))
    node, nn_a, nn_b = _embed(node_type, emb_pad, p["conv0"]["W1"])
    zeros_nc = jnp.zeros((N, 32), jnp.float32)
    res = None
    for i in range(N_CONV):
        cf_a, cf_b = _conv_sc(nn_a, nn_b, hs[2 * i], hs[2 * i + 1],
                              src2, dst2, zeros_nc)
        ci = p["conv%d" % i]
        if i < N_CONV - 1:
            w1n = p["conv%d" % (i + 1)]["W1"]
            node, nn_a, nn_b = _update(
                cf_a, cf_b, node, ci["W2"], ci["b2"].reshape(1, DIM),
                ci["W3"], ci["b3"].reshape(1, DIM), w1n)
        else:
            res = _head(
                cf_a, cf_b, node, ci["W2"], ci["b2"].reshape(1, DIM),
                ci["W3"], ci["b3"].reshape(1, DIM),
                p["d1_W"], p["d1_b"].reshape(1, DIM),
                p["d2_W"].reshape(1, DIM), p["d2_b"].reshape(1, 1))
    res3 = res.reshape(N // IDX_W, IDX_W, 16)
    gid2 = graph_ids.reshape(N // IDX_W, IDX_W)
    pooled = _pool_sc(res3, gid2, jnp.zeros((G_PAD, 16), jnp.float32))
    return pooled[:N_GRAPHS, 0:1] / jnp.maximum(pooled[:N_GRAPHS, 1:2], 1.0)
